# baseline (speedup 1.0000x reference)
import functools

import jax
import jax.numpy as jnp
from jax import lax
from jax.experimental import pallas as pl
from jax.experimental.pallas import tpu as pltpu
from jax.experimental.pallas import tpu_sc as plsc

_B, _C, _H, _W = 128, 96, 32, 32
_N = _H * _W
_K = 256
_ROWS = _B * _C
_NW = 32
_RPW = _ROWS // _NW
_L = 16
_SQRT_HALF = 0.7071067811865476
_IMGS = 4


def _gelu(x):
    return 0.5 * x * (1.0 + jax.lax.erf(x * _SQRT_HALF))


def _dot(a, b):
    return jax.lax.dot_general(
        a, b, (((1,), (0,)), ((), ())),
        preferred_element_type=jnp.float32,
        precision=jax.lax.Precision.HIGHEST,
    )


def _tc_body(x_ref, g_ref, w1_ref, b1_ref, w2_ref, b2_ref, w3_ref, b3_ref,
             w4_ref, out_ref):
    for i in range(_IMGS):
        x = x_ref[i].reshape(_C, _N)
        h = _gelu(_dot(w1_ref[...], x) + b1_ref[...])
        h = _gelu(_dot(w2_ref[...], h) + b2_ref[...])
        h = _gelu(_dot(w3_ref[...], h) + b3_ref[...])
        s = _dot(w4_ref[...], h) + g_ref[i]
        keep = s >= 0.0
        out_ref[pl.ds(i * _C, _C), :] = jnp.where(keep, jnp.maximum(x, 0.0), 0.0)


def _vsort(v):
    return plsc.sort_key_val(v, v)[0]


def _vrev(v):
    return lax.rev(v, (0,))


def _clean_asc(c):
    m = len(c)
    s = m // 2
    while s >= 1:
        d = list(c)
        for blk in range(0, m, 2 * s):
            for i in range(blk, blk + s):
                d[i] = jnp.minimum(c[i], c[i + s])
                d[i + s] = jnp.maximum(c[i], c[i + s])
        c = d
        s //= 2
    return [_vsort(x) for x in c]


def _vmerge(a, b):
    m = len(a)
    return _clean_asc(list(a) + [_vrev(b[m - 1 - i]) for i in range(m)])


def _merge_discard_high(a, b):
    m = len(a)
    c = [jnp.maximum(a[i], _vrev(b[m - 1 - i])) for i in range(m)]
    return _clean_asc(c)


def _sort256(vregs):
    runs = [[_vsort(v)] for v in vregs]
    while len(runs) > 1:
        runs = [_vmerge(runs[i], runs[i + 1]) for i in range(0, len(runs), 2)]
    return runs[0]


def _topk_row(load):
    w = [load(c) for c in range(64)]
    chunks = [_sort256(w[c * 16:(c + 1) * 16]) for c in range(4)]
    t = _merge_discard_high(chunks[0], chunks[1])
    u = _merge_discard_high(chunks[2], chunks[3])
    return _merge_discard_high(t, u)


def _sc_topk(ct):
    mesh = plsc.VectorSubcoreMesh(core_axis_name="c", subcore_axis_name="s")

    @functools.partial(
        pl.kernel, mesh=mesh,
        out_type=jax.ShapeDtypeStruct((_ROWS, _K), jnp.float32),
        scratch_types=[
            pltpu.VMEM((2, _N), jnp.float32),
            pltpu.VMEM((2, _K), jnp.float32),
            pltpu.SemaphoreType.DMA,
            pltpu.SemaphoreType.DMA,
            pltpu.SemaphoreType.DMA,
            pltpu.SemaphoreType.DMA,
        ],
        compiler_params=pltpu.CompilerParams(needs_layout_passes=False),
    )
    def k(ct_hbm, out_hbm, buf, obuf, si0, si1, so0, so1):
        wid = lax.axis_index("s") * 2 + lax.axis_index("c")
        base = wid * _RPW
        sis = (si0, si1)
        sos = (so0, so1)

        pltpu.make_async_copy(ct_hbm.at[base], buf.at[0], si0).start()
        pltpu.make_async_copy(ct_hbm.at[base + 1], buf.at[1], si1).start()

        def body(ib, carry):
            for par in range(2):
                row = base + 2 * ib + par
                pltpu.make_async_copy(ct_hbm.at[row], buf.at[par],
                                      sis[par]).wait()
                res = _topk_row(lambda c: buf[par, pl.ds(c * _L, _L)])

                @pl.when(ib > 0)
                def _():
                    pltpu.make_async_copy(obuf.at[par], out_hbm.at[row],
                                          sos[par]).wait()

                for i in range(16):
                    obuf[par, pl.ds(i * _L, _L)] = _vrev(res[15 - i])
                pltpu.make_async_copy(obuf.at[par], out_hbm.at[row],
                                      sos[par]).start()

                @pl.when(row + 2 < base + _RPW)
                def _():
                    pltpu.make_async_copy(ct_hbm.at[row + 2], buf.at[par],
                                          sis[par]).start()
            return carry

        lax.fori_loop(0, _RPW // 2, body, 0)
        pltpu.make_async_copy(obuf.at[0], out_hbm.at[base], so0).wait()
        pltpu.make_async_copy(obuf.at[1], out_hbm.at[base], so1).wait()

    return k(ct)


def kernel(x, W1, b1, W2, b2, W3, b3, W4, b4):
    b, c, h, w = x.shape
    n = h * w
    g = jax.random.gumbel(jax.random.key(42), (b, n, 2), dtype=jnp.float32)
    gdiff = (g[:, :, 0] - g[:, :, 1] + (b4[0] - b4[1])).reshape(b, 1, n)
    w4d = (W4[0] - W4[1]).reshape(1, c // 4)

    ct = pl.pallas_call(
        _tc_body,
        grid=(b // _IMGS,),
        in_specs=[
            pl.BlockSpec((_IMGS, c, h, w), lambda i: (i, 0, 0, 0)),
            pl.BlockSpec((_IMGS, 1, n), lambda i: (i, 0, 0)),
            pl.BlockSpec((c, c), lambda i: (0, 0)),
            pl.BlockSpec((c, 1), lambda i: (0, 0)),
            pl.BlockSpec((c // 2, c), lambda i: (0, 0)),
            pl.BlockSpec((c // 2, 1), lambda i: (0, 0)),
            pl.BlockSpec((c // 4, c // 2), lambda i: (0, 0)),
            pl.BlockSpec((c // 4, 1), lambda i: (0, 0)),
            pl.BlockSpec((1, c // 4), lambda i: (0, 0)),
        ],
        out_specs=pl.BlockSpec((_IMGS * c, n), lambda i: (i, 0)),
        out_shape=jax.ShapeDtypeStruct((b * c, n), jnp.float32),
    )(x, gdiff, W1, b1.reshape(c, 1), W2, b2.reshape(c // 2, 1),
      W3, b3.reshape(c // 4, 1), w4d)

    out = _sc_topk(ct)
    return out.reshape(b, c, _H // 2, _W // 2)


# speedup vs baseline: 1.2860x; 1.2860x over previous
import functools

import jax
import jax.numpy as jnp
from jax import lax
from jax.experimental import pallas as pl
from jax.experimental.pallas import tpu as pltpu
from jax.experimental.pallas import tpu_sc as plsc

_B, _C, _H, _W = 128, 96, 32, 32
_N = _H * _W
_K = 256
_ROWS = _B * _C
_NW = 32
_RPW = _ROWS // _NW
_L = 16
_SQRT_HALF = 0.7071067811865476
_IMGS = 8


def _gelu(x):
    return 0.5 * x * (1.0 + jax.lax.erf(x * _SQRT_HALF))


def _dot(a, b):
    return jax.lax.dot_general(
        a, b, (((1,), (0,)), ((), ())),
        preferred_element_type=jnp.float32,
        precision=jax.lax.Precision.HIGHEST,
    )


def _tc_body(x_ref, g_ref, w1_ref, b1_ref, w2_ref, b2_ref, w3_ref, b3_ref,
             w4_ref, out_ref):
    for i in range(_IMGS):
        x = x_ref[i]
        h = _gelu(_dot(w1_ref[...], x) + b1_ref[...])
        h = _gelu(_dot(w2_ref[...], h) + b2_ref[...])
        h = _gelu(_dot(w3_ref[...], h) + b3_ref[...])
        s = _dot(w4_ref[...], h) + g_ref[i]
        keep = s >= 0.0
        out_ref[pl.ds(i * _C, _C), :] = jnp.where(keep, jnp.maximum(x, 0.0), 0.0)


def _vsort(v):
    return plsc.sort_key_val(v, v)[0]


def _vrev(v):
    return lax.rev(v, (0,))


def _clean_asc(c):
    m = len(c)
    s = m // 2
    while s >= 1:
        d = list(c)
        for blk in range(0, m, 2 * s):
            for i in range(blk, blk + s):
                d[i] = jnp.minimum(c[i], c[i + s])
                d[i + s] = jnp.maximum(c[i], c[i + s])
        c = d
        s //= 2
    return [_vsort(x) for x in c]


def _vmerge(a, b):
    m = len(a)
    return _clean_asc(list(a) + [_vrev(b[m - 1 - i]) for i in range(m)])


def _merge_discard_high(a, b):
    m = len(a)
    c = [jnp.maximum(a[i], _vrev(b[m - 1 - i])) for i in range(m)]
    return _clean_asc(c)


def _sort256(vregs):
    runs = [[_vsort(v)] for v in vregs]
    while len(runs) > 1:
        runs = [_vmerge(runs[i], runs[i + 1]) for i in range(0, len(runs), 2)]
    return runs[0]


def _topk_row(load):
    w = [load(c) for c in range(64)]
    chunks = [_sort256(w[c * 16:(c + 1) * 16]) for c in range(4)]
    t = _merge_discard_high(chunks[0], chunks[1])
    u = _merge_discard_high(chunks[2], chunks[3])
    return _merge_discard_high(t, u)


def _sc_topk(ct):
    mesh = plsc.VectorSubcoreMesh(core_axis_name="c", subcore_axis_name="s")

    @functools.partial(
        pl.kernel, mesh=mesh,
        out_type=jax.ShapeDtypeStruct((_ROWS, _K), jnp.float32),
        scratch_types=[
            pltpu.VMEM((2, _N), jnp.float32),
            pltpu.VMEM((2, _K), jnp.float32),
            pltpu.SemaphoreType.DMA,
            pltpu.SemaphoreType.DMA,
            pltpu.SemaphoreType.DMA,
            pltpu.SemaphoreType.DMA,
        ],
        compiler_params=pltpu.CompilerParams(needs_layout_passes=False),
    )
    def k(ct_hbm, out_hbm, buf, obuf, si0, si1, so0, so1):
        wid = lax.axis_index("s") * 2 + lax.axis_index("c")
        base = wid * _RPW
        sis = (si0, si1)
        sos = (so0, so1)

        pltpu.make_async_copy(ct_hbm.at[base], buf.at[0], si0).start()
        pltpu.make_async_copy(ct_hbm.at[base + 1], buf.at[1], si1).start()

        def body(ib, carry):
            for par in range(2):
                row = base + 2 * ib + par
                pltpu.make_async_copy(ct_hbm.at[row], buf.at[par],
                                      sis[par]).wait()
                res = _topk_row(lambda c: buf[par, pl.ds(c * _L, _L)])

                @pl.when(ib > 0)
                def _():
                    pltpu.make_async_copy(obuf.at[par], out_hbm.at[row],
                                          sos[par]).wait()

                for i in range(16):
                    obuf[par, pl.ds(i * _L, _L)] = _vrev(res[15 - i])
                pltpu.make_async_copy(obuf.at[par], out_hbm.at[row],
                                      sos[par]).start()

                @pl.when(row + 2 < base + _RPW)
                def _():
                    pltpu.make_async_copy(ct_hbm.at[row + 2], buf.at[par],
                                          sis[par]).start()
            return carry

        lax.fori_loop(0, _RPW // 2, body, 0)
        pltpu.make_async_copy(obuf.at[0], out_hbm.at[base], so0).wait()
        pltpu.make_async_copy(obuf.at[1], out_hbm.at[base], so1).wait()

    return k(ct)


def kernel(x, W1, b1, W2, b2, W3, b3, W4, b4):
    b, c, h, w = x.shape
    n = h * w
    xr = x.reshape(b, c, n)
    g = jax.random.gumbel(jax.random.key(42), (b, n, 2), dtype=jnp.float32)
    gdiff = (g[:, :, 0] - g[:, :, 1] + (b4[0] - b4[1])).reshape(b, 1, n)
    w4d = (W4[0] - W4[1]).reshape(1, c // 4)

    ct = pl.pallas_call(
        _tc_body,
        grid=(b // _IMGS,),
        in_specs=[
            pl.BlockSpec((_IMGS, c, n), lambda i: (i, 0, 0)),
            pl.BlockSpec((_IMGS, 1, n), lambda i: (i, 0, 0)),
            pl.BlockSpec((c, c), lambda i: (0, 0)),
            pl.BlockSpec((c, 1), lambda i: (0, 0)),
            pl.BlockSpec((c // 2, c), lambda i: (0, 0)),
            pl.BlockSpec((c // 2, 1), lambda i: (0, 0)),
            pl.BlockSpec((c // 4, c // 2), lambda i: (0, 0)),
            pl.BlockSpec((c // 4, 1), lambda i: (0, 0)),
            pl.BlockSpec((1, c // 4), lambda i: (0, 0)),
        ],
        out_specs=pl.BlockSpec((_IMGS * c, n), lambda i: (i, 0)),
        out_shape=jax.ShapeDtypeStruct((b * c, n), jnp.float32),
    )(xr, gdiff, W1, b1.reshape(c, 1), W2, b2.reshape(c // 2, 1),
      W3, b3.reshape(c // 4, 1), w4d)

    out = _sc_topk(ct)
    return out.reshape(b, c, _H // 2, _W // 2)
